# Initial kernel scaffold; baseline (speedup 1.0000x reference)
import jax
import jax.numpy as jnp
from jax.experimental import pallas as pl


def kernel(user, item, label, eu_gmf, ei_gmf, eu_mlp, ei_mlp, W1, b1, W2, b2, W3, b3, Wp, bp):
    raise NotImplementedError("write your pallas kernel here")


# speedup vs baseline: 3.6776x; 3.6776x over previous
import functools

import jax
import jax.numpy as jnp
from jax import lax
from jax.experimental import pallas as pl
from jax.experimental.pallas import tpu as pltpu
from jax.experimental.pallas import tpu_sc as plsc

B = 16384
DG = 32
DM = 32
H1, H2, H3 = 64, 32, 16

NC, NS = 2, 16
NW = NC * NS
BPW = B // NW
CH = 128
NCH = BPW // CH


def _sc_gather(user, item, eu_gmf, ei_gmf, eu_mlp, ei_mlp):
    mesh = plsc.VectorSubcoreMesh(core_axis_name="c", subcore_axis_name="s")
    row = jax.ShapeDtypeStruct((B, DG), jnp.float32)

    @functools.partial(
        pl.kernel,
        out_type=(row, row, row, row),
        mesh=mesh,
        scratch_types=[
            pltpu.VMEM((NCH, CH), jnp.int32),
            pltpu.VMEM((NCH, CH), jnp.int32),
            pltpu.VMEM((BPW, DG), jnp.float32),
            pltpu.VMEM((BPW, DG), jnp.float32),
            pltpu.VMEM((BPW, DG), jnp.float32),
            pltpu.VMEM((BPW, DG), jnp.float32),
            pltpu.SemaphoreType.DMA,
        ],
    )
    def gather_kernel(user_h, item_h, eug_h, eig_h, eum_h, eim_h,
                      out_ug, out_ig, out_um, out_im,
                      uidx, iidx, bug, big, bum, bim, sem):
        wid = lax.axis_index("s") * NC + lax.axis_index("c")
        base = wid * BPW
        for j in range(NCH):
            sl = pl.ds(base + j * CH, CH)
            pltpu.sync_copy(user_h.at[sl], uidx.at[j])
            pltpu.sync_copy(item_h.at[sl], iidx.at[j])
        copies = []
        for j in range(NCH):
            dst = pl.ds(j * CH, CH)
            copies.append(pltpu.async_copy(eug_h.at[uidx.at[j]], bug.at[dst], sem))
            copies.append(pltpu.async_copy(eig_h.at[iidx.at[j]], big.at[dst], sem))
            copies.append(pltpu.async_copy(eum_h.at[uidx.at[j]], bum.at[dst], sem))
            copies.append(pltpu.async_copy(eim_h.at[iidx.at[j]], bim.at[dst], sem))
        for c in copies:
            c.wait()
        out_sl = pl.ds(base, BPW)
        pltpu.sync_copy(bug, out_ug.at[out_sl])
        pltpu.sync_copy(big, out_ig.at[out_sl])
        pltpu.sync_copy(bum, out_um.at[out_sl])
        pltpu.sync_copy(bim, out_im.at[out_sl])

    return gather_kernel(user, item, eu_gmf, ei_gmf, eu_mlp, ei_mlp)


BLK = 1024
GRID = B // BLK


def _mlp_body(egu, egi, emu, emi, lab,
              w1a, w1b, b1, w2, b2, w3, b3, wpa, wpb, bp,
              m1_r, m2_r, m3_r, pred_r, loss_r):
    i = pl.program_id(0)
    dot = functools.partial(
        lax.dot_general,
        dimension_numbers=(((1,), (1,)), ((), ())),
        preferred_element_type=jnp.float32,
    )
    m1 = jnp.maximum(dot(emu[...], w1a[...]) + dot(emi[...], w1b[...]) + b1[...], 0.0)
    m2 = jnp.maximum(dot(m1, w2[...]) + b2[...], 0.0)
    m3 = jnp.maximum(dot(m2, w3[...]) + b3[...], 0.0)
    gmf = egu[...] * egi[...]
    s = dot(gmf, wpa[...]) + dot(m3, wpb[...]) + bp[...]
    pred = jax.nn.sigmoid(s)
    m1_r[...] = m1
    m2_r[...] = m2
    m3_r[...] = m3
    pred_r[...] = pred
    p = jnp.clip(pred, 1e-7, 1.0 - 1e-7)
    y = lab[...].astype(jnp.float32)
    part = jnp.sum(-(y * jnp.log(p) + (1.0 - y) * jnp.log(1.0 - p)))

    @pl.when(i == 0)
    def _():
        loss_r[0, 0] = part

    @pl.when(i > 0)
    def _():
        loss_r[0, 0] += part

    @pl.when(i == GRID - 1)
    def _():
        loss_r[0, 0] = loss_r[0, 0] / B


def _tc_mlp(egu, egi, emu, emi, lab2, W1a, W1b, b1, W2, b2, W3, b3, Wpa, Wpb, bp):
    bspec = lambda d: pl.BlockSpec((BLK, d), lambda i: (i, 0))
    wspec = lambda r, c: pl.BlockSpec((r, c), lambda i: (0, 0))
    return pl.pallas_call(
        _mlp_body,
        grid=(GRID,),
        in_specs=[
            bspec(DG), bspec(DG), bspec(DM), bspec(DM), bspec(1),
            wspec(H1, DM), wspec(H1, DM), wspec(1, H1),
            wspec(H2, H1), wspec(1, H2),
            wspec(H3, H2), wspec(1, H3),
            wspec(1, DG), wspec(1, H3), wspec(1, 1),
        ],
        out_specs=[
            bspec(H1), bspec(H2), bspec(H3), bspec(1),
            pl.BlockSpec(memory_space=pltpu.SMEM, block_shape=(1, 1),
                         index_map=lambda i: (0, 0)),
        ],
        out_shape=[
            jax.ShapeDtypeStruct((B, H1), jnp.float32),
            jax.ShapeDtypeStruct((B, H2), jnp.float32),
            jax.ShapeDtypeStruct((B, H3), jnp.float32),
            jax.ShapeDtypeStruct((B, 1), jnp.float32),
            jax.ShapeDtypeStruct((1, 1), jnp.float32),
        ],
    )(egu, egi, emu, emi, lab2, W1a, W1b, b1, W2, b2, W3, b3, Wpa, Wpb, bp)


def kernel(user, item, label, eu_gmf, ei_gmf, eu_mlp, ei_mlp,
           W1, b1, W2, b2, W3, b3, Wp, bp):
    egu = jnp.take(eu_gmf, user, axis=0)
    egi = jnp.take(ei_gmf, item, axis=0)
    emu = jnp.take(eu_mlp, user, axis=0)
    emi = jnp.take(ei_mlp, item, axis=0)
    W1a, W1b = W1[:, :DM], W1[:, DM:]
    Wpa, Wpb = Wp[:, :DG], Wp[:, DG:]
    m1, m2, m3, pred2, loss = _tc_mlp(
        egu, egi, emu, emi, label.reshape(B, 1),
        W1a, W1b, b1.reshape(1, H1), W2, b2.reshape(1, H2),
        W3, b3.reshape(1, H3), Wpa, Wpb, bp.reshape(1, 1),
    )
    return (loss[0, 0], m1, m2, m3, pred2.reshape(-1))
